# baseline (speedup 1.0000x reference)
import functools

import jax
import jax.numpy as jnp
from jax import lax
from jax.experimental import pallas as pl
from jax.experimental.pallas import tpu as pltpu
from jax.experimental.pallas import tpu_sc as plsc

NN = 10000
EE = 160000
DD = 128
DW = 144
NREL = 3
NC = 2
NS = 16
CHUNK = 128
CH = 80
EPAD = NS * CH * CHUNK
HALF = NN // NC
ACC_H = 5001
OUT_H = 5000
CNTW = 16
KPT = 3


def _seg_body(table, src_h, dst_h, zeros_h, g_out, src_v, dst_v,
              rows0_v, rows1_v, zero_v, acc, sem0, sem1):
    c = lax.axis_index("c")
    s = lax.axis_index("s")

    pltpu.sync_copy(zeros_h, zero_v)

    def row_base(k):
        return jnp.minimum((s + NS * k) * CHUNK, OUT_H - CHUNK)

    def per_etype(e, carry):
        def zero_step(k, cc):
            pltpu.sync_copy(zero_v, acc.at[pl.ds(row_base(k), CHUNK)])
            return cc

        lax.fori_loop(0, KPT, zero_step, 0)
        plsc.subcore_barrier()

        pltpu.sync_copy(
            src_h.at[pl.ds((e * NS + s) * CH * CHUNK, CH * CHUNK)], src_v)
        pltpu.sync_copy(dst_h.at[pl.ds(((c * NREL + e) * NS + s) * CH, CH)],
                        dst_v)

        pltpu.async_copy(
            table.at[src_v.at[pl.ds(0, CHUNK)]], rows0_v, sem0)

        def pair_step(p, cc):
            j0 = 2 * p
            j1 = 2 * p + 1
            pltpu.async_copy(
                table.at[src_v.at[pl.ds(j1 * CHUNK, CHUNK)]], rows1_v, sem1)
            pltpu.make_async_copy(
                table.at[src_v.at[pl.ds(j0 * CHUNK, CHUNK)]], rows0_v,
                sem0).wait()
            pltpu.sync_copy(rows0_v, acc.at[dst_v.at[j0]], add=True)

            @pl.when(p + 1 < CH // 2)
            def _():
                pltpu.async_copy(
                    table.at[src_v.at[pl.ds((j1 + 1) * CHUNK, CHUNK)]],
                    rows0_v, sem0)

            pltpu.make_async_copy(
                table.at[src_v.at[pl.ds(j1 * CHUNK, CHUNK)]], rows1_v,
                sem1).wait()
            pltpu.sync_copy(rows1_v, acc.at[dst_v.at[j1]], add=True)
            return cc

        lax.fori_loop(0, CH // 2, pair_step, 0)
        plsc.subcore_barrier()

        def dump_step(k, cc):
            base = row_base(k)
            pltpu.sync_copy(acc.at[pl.ds(base, CHUNK)],
                            g_out.at[e, c, pl.ds(base, CHUNK)])
            return cc

        lax.fori_loop(0, KPT, dump_step, 0)
        plsc.subcore_barrier()
        return carry

    lax.fori_loop(0, NREL, per_etype, 0)


def _make_seg_kernel():
    mesh = plsc.VectorSubcoreMesh(core_axis_name="c", subcore_axis_name="s")
    out_type = jax.ShapeDtypeStruct((NREL, NC, OUT_H, DW), jnp.float32)
    scratch = [
        pltpu.VMEM((CH * CHUNK,), jnp.int32),
        pltpu.VMEM((CH, CHUNK), jnp.int32),
        pltpu.VMEM((CHUNK, DW), jnp.float32),
        pltpu.VMEM((CHUNK, DW), jnp.float32),
        pltpu.VMEM((CHUNK, DW), jnp.float32),
        pltpu.VMEM_SHARED((ACC_H, DW), jnp.float32),
        pltpu.SemaphoreType.DMA,
        pltpu.SemaphoreType.DMA,
    ]
    return pl.kernel(
        _seg_body,
        out_type=out_type,
        mesh=mesh,
        scratch_types=scratch,
        compiler_params=pltpu.CompilerParams(use_tc_tiling_on_sc=False),
    )


_seg = _make_seg_kernel()

BR = 1000
NBLK = HALF // BR


def _tc_layer1_body(gp, w, b, out, cnt_out):
    g = gp[:, 0]
    cnt = g[:, :, DD]
    denom = jnp.maximum(cnt, 1.0)
    ind = (cnt > 0.0).astype(jnp.float32)
    acc = jnp.zeros((BR, DD), jnp.float32)
    for e in range(NREL):
        gn = g[e, :, :DD] * (1.0 / denom[e])[:, None]
        acc = acc + jnp.dot(gn, w[e], preferred_element_type=jnp.float32)
        acc = acc + ind[e][:, None] * b[e][None, :]
    acc = jnp.maximum(acc, 0.0)
    lane = lax.broadcasted_iota(jnp.int32, (BR, DW - DD), 1)
    tail = (lane == 0).astype(jnp.float32)
    out[...] = jnp.concatenate([acc, tail], axis=1)
    cnt_out[...] = jnp.broadcast_to(cnt[:, :, None], (NREL, BR, CNTW))


def _tc_layer2_body(gp, cp, w, b, out):
    g = gp[:, 0]
    cnt = cp[:, :, 0]
    denom = jnp.maximum(cnt, 1.0)
    ind = (cnt > 0.0).astype(jnp.float32)
    acc = jnp.zeros((BR, DD), jnp.float32)
    for e in range(NREL):
        gn = g[e, :, :DD] * (1.0 / denom[e])[:, None]
        acc = acc + jnp.dot(gn, w[e], preferred_element_type=jnp.float32)
        acc = acc + ind[e][:, None] * b[e][None, :]
    out[...] = acc


_g_spec = pl.BlockSpec((NREL, 1, BR, DW), lambda i: (0, i // NBLK, i % NBLK, 0))
_w_spec = pl.BlockSpec((NREL, DD, DD), lambda i: (0, 0, 0))
_b_spec = pl.BlockSpec((NREL, DD), lambda i: (0, 0))
_cnt_spec = pl.BlockSpec((NREL, BR, CNTW), lambda i: (0, i, 0))

_tc_layer1 = pl.pallas_call(
    _tc_layer1_body,
    grid=(NN // BR,),
    in_specs=[_g_spec, _w_spec, _b_spec],
    out_specs=[pl.BlockSpec((BR, DW), lambda i: (i, 0)), _cnt_spec],
    out_shape=[
        jax.ShapeDtypeStruct((NN, DW), jnp.float32),
        jax.ShapeDtypeStruct((NREL, NN, CNTW), jnp.float32),
    ],
)

_tc_layer2 = pl.pallas_call(
    _tc_layer2_body,
    grid=(NN // BR,),
    in_specs=[_g_spec, _cnt_spec, _w_spec, _b_spec],
    out_specs=pl.BlockSpec((BR, DD), lambda i: (i, 0)),
    out_shape=jax.ShapeDtypeStruct((NN, DD), jnp.float32),
)


def kernel(feat, edge_index_rel0, edge_index_rel1, edge_index_rel2,
           W1_rel0, b1_rel0, W1_rel1, b1_rel1, W1_rel2, b1_rel2,
           W2_rel0, b2_rel0, W2_rel1, b2_rel1, W2_rel2, b2_rel2):
    src = jnp.stack([edge_index_rel0[0], edge_index_rel1[0], edge_index_rel2[0]])
    dst = jnp.stack([edge_index_rel0[1], edge_index_rel1[1], edge_index_rel2[1]])
    pad = EPAD - EE
    srcp = jnp.concatenate(
        [src, jnp.zeros((NREL, pad), jnp.int32)], axis=1
    ).reshape(NREL * NS * CH * CHUNK)
    dstp_cores = []
    dpad = jnp.concatenate([dst, jnp.full((NREL, pad), NN, jnp.int32)], axis=1)
    for c in range(NC):
        dl = dpad - c * HALF
        dl = jnp.where((dl >= 0) & (dl < HALF), dl, HALF)
        dstp_cores.append(dl.reshape(NREL * NS * CH, CHUNK))
    dstp = jnp.concatenate(dstp_cores)

    zeros_w = jnp.zeros((CHUNK, DW), jnp.float32)
    tail = jnp.concatenate(
        [jnp.ones((NN, 1), jnp.float32),
         jnp.zeros((NN, DW - DD - 1), jnp.float32)], axis=1)
    feat_w = jnp.concatenate([feat, tail], axis=1)

    w1 = jnp.stack([W1_rel0, W1_rel1, W1_rel2])
    b1 = jnp.stack([b1_rel0, b1_rel1, b1_rel2])
    w2 = jnp.stack([W2_rel0, W2_rel1, W2_rel2])
    b2 = jnp.stack([b2_rel0, b2_rel1, b2_rel2])

    g1 = _seg(feat_w, srcp, dstp, zeros_w)
    h1w, cnts = _tc_layer1(g1, w1, b1)
    g2 = _seg(h1w, srcp, dstp, zeros_w)
    out = _tc_layer2(g2, cnts, w2, b2)
    return out


# speedup vs baseline: 1.6323x; 1.6323x over previous
import functools

import jax
import jax.numpy as jnp
from jax import lax
from jax.experimental import pallas as pl
from jax.experimental.pallas import tpu as pltpu
from jax.experimental.pallas import tpu_sc as plsc

NN = 10000
EE = 160000
DD = 128
NREL = 3
NC = 2
NS = 16
CHUNK = 128
CH = 80
EPAD = NS * CH * CHUNK
HALF = NN // NC
ACC_H = 5001
OUT_H = 5000
CNTW = 16
ZCH = 40
KPT = 3


def _seg_body(table, src_h, dst_h, ones_h, zeros_h, zeros16_h,
              g_out, cnt_out, src_v, dst_v, rows0_v, rows1_v, ones_v, zero_v,
              zero16_v, acc, cnt, sem0, sem1):
    c = lax.axis_index("c")
    s = lax.axis_index("s")

    pltpu.sync_copy(zeros_h, zero_v)
    pltpu.sync_copy(ones_h, ones_v)
    pltpu.sync_copy(zeros16_h, zero16_v)

    def row_base(k):
        return jnp.minimum((s + NS * k) * CHUNK, OUT_H - CHUNK)

    def per_etype(e, carry):
        def zero_step(k, cc):
            base = row_base(k)
            pltpu.sync_copy(zero_v, acc.at[pl.ds(base, CHUNK)])
            pltpu.sync_copy(zero16_v, cnt.at[pl.ds(base, CHUNK)])
            return cc

        lax.fori_loop(0, KPT, zero_step, 0)
        plsc.subcore_barrier()

        pltpu.sync_copy(
            src_h.at[pl.ds((e * NS + s) * CH * CHUNK, CH * CHUNK)], src_v)
        pltpu.sync_copy(dst_h.at[pl.ds(((c * NREL + e) * NS + s) * CH, CH)],
                        dst_v)

        pltpu.async_copy(
            table.at[src_v.at[pl.ds(0, CHUNK)]], rows0_v, sem0)

        def pair_step(p, cc):
            j0 = 2 * p
            j1 = 2 * p + 1
            pltpu.async_copy(
                table.at[src_v.at[pl.ds(j1 * CHUNK, CHUNK)]], rows1_v, sem1)
            pltpu.make_async_copy(
                table.at[src_v.at[pl.ds(j0 * CHUNK, CHUNK)]], rows0_v,
                sem0).wait()
            pltpu.sync_copy(rows0_v, acc.at[dst_v.at[j0]], add=True)
            pltpu.sync_copy(ones_v, cnt.at[dst_v.at[j0]], add=True)

            @pl.when(p + 1 < CH // 2)
            def _():
                pltpu.async_copy(
                    table.at[src_v.at[pl.ds((j1 + 1) * CHUNK, CHUNK)]],
                    rows0_v, sem0)

            pltpu.make_async_copy(
                table.at[src_v.at[pl.ds(j1 * CHUNK, CHUNK)]], rows1_v,
                sem1).wait()
            pltpu.sync_copy(rows1_v, acc.at[dst_v.at[j1]], add=True)
            pltpu.sync_copy(ones_v, cnt.at[dst_v.at[j1]], add=True)
            return cc

        lax.fori_loop(0, CH // 2, pair_step, 0)
        plsc.subcore_barrier()

        def dump_step(k, cc):
            base = row_base(k)
            pltpu.sync_copy(acc.at[pl.ds(base, CHUNK)],
                            g_out.at[e, c, pl.ds(base, CHUNK)])
            pltpu.sync_copy(cnt.at[pl.ds(base, CHUNK)],
                            cnt_out.at[e, c, pl.ds(base, CHUNK)])
            return cc

        lax.fori_loop(0, KPT, dump_step, 0)
        plsc.subcore_barrier()
        return carry

    lax.fori_loop(0, NREL, per_etype, 0)


def _make_seg_kernel():
    mesh = plsc.VectorSubcoreMesh(core_axis_name="c", subcore_axis_name="s")
    out_type = (
        jax.ShapeDtypeStruct((NREL, NC, OUT_H, DD), jnp.bfloat16),
        jax.ShapeDtypeStruct((NREL, NC, OUT_H, CNTW), jnp.float32),
    )
    scratch = [
        pltpu.VMEM((CH * CHUNK,), jnp.int32),
        pltpu.VMEM((CH, CHUNK), jnp.int32),
        pltpu.VMEM((CHUNK, DD), jnp.bfloat16),
        pltpu.VMEM((CHUNK, DD), jnp.bfloat16),
        pltpu.VMEM((CHUNK, CNTW), jnp.float32),
        pltpu.VMEM((CHUNK, DD), jnp.bfloat16),
        pltpu.VMEM((CHUNK, CNTW), jnp.float32),
        pltpu.VMEM_SHARED((ACC_H, DD), jnp.bfloat16),
        pltpu.VMEM_SHARED((ACC_H, CNTW), jnp.float32),
        pltpu.SemaphoreType.DMA,
        pltpu.SemaphoreType.DMA,
    ]
    return pl.kernel(
        _seg_body,
        out_type=out_type,
        mesh=mesh,
        scratch_types=scratch,
        compiler_params=pltpu.CompilerParams(use_tc_tiling_on_sc=False),
    )


_seg = _make_seg_kernel()

BR = 1000
NBLK = HALF // BR


def _tc_body(relu, gp, cp, w, b, out):
    g = gp[:, 0].astype(jnp.float32)
    cnt = cp[:, 0, :, 0]
    denom = jnp.maximum(cnt, 1.0)
    ind = (cnt > 0.0).astype(jnp.float32)
    acc = jnp.zeros((BR, DD), jnp.float32)
    for e in range(NREL):
        gn = g[e] * (1.0 / denom[e])[:, None]
        acc = acc + jnp.dot(gn, w[e], preferred_element_type=jnp.float32)
        acc = acc + ind[e][:, None] * b[e][None, :]
    if relu:
        acc = jnp.maximum(acc, 0.0)
    out[...] = acc.astype(out.dtype)


def _make_tc_layer(relu, out_dtype):
    return pl.pallas_call(
        functools.partial(_tc_body, relu),
        grid=(NN // BR,),
        in_specs=[
            pl.BlockSpec((NREL, 1, BR, DD),
                         lambda i: (0, i // NBLK, i % NBLK, 0)),
            pl.BlockSpec((NREL, 1, BR, CNTW),
                         lambda i: (0, i // NBLK, i % NBLK, 0)),
            pl.BlockSpec((NREL, DD, DD), lambda i: (0, 0, 0)),
            pl.BlockSpec((NREL, DD), lambda i: (0, 0)),
        ],
        out_specs=pl.BlockSpec((BR, DD), lambda i: (i, 0)),
        out_shape=jax.ShapeDtypeStruct((NN, DD), out_dtype),
    )


_tc_relu = _make_tc_layer(True, jnp.bfloat16)
_tc_lin = _make_tc_layer(False, jnp.float32)


def kernel(feat, edge_index_rel0, edge_index_rel1, edge_index_rel2,
           W1_rel0, b1_rel0, W1_rel1, b1_rel1, W1_rel2, b1_rel2,
           W2_rel0, b2_rel0, W2_rel1, b2_rel1, W2_rel2, b2_rel2):
    src = jnp.stack([edge_index_rel0[0], edge_index_rel1[0], edge_index_rel2[0]])
    dst = jnp.stack([edge_index_rel0[1], edge_index_rel1[1], edge_index_rel2[1]])
    pad = EPAD - EE
    srcp = jnp.concatenate(
        [src, jnp.zeros((NREL, pad), jnp.int32)], axis=1
    ).reshape(NREL * NS * CH * CHUNK)
    dstp_cores = []
    dpad = jnp.concatenate([dst, jnp.full((NREL, pad), NN, jnp.int32)], axis=1)
    for c in range(NC):
        dl = dpad - c * HALF
        dl = jnp.where((dl >= 0) & (dl < HALF), dl, HALF)
        dstp_cores.append(dl.reshape(NREL * NS * CH, CHUNK))
    dstp = jnp.concatenate(dstp_cores)

    ones16 = jnp.ones((CHUNK, CNTW), jnp.float32)
    zeros128 = jnp.zeros((CHUNK, DD), jnp.bfloat16)
    zeros16 = jnp.zeros((CHUNK, CNTW), jnp.float32)
    feat = feat.astype(jnp.bfloat16)

    w1 = jnp.stack([W1_rel0, W1_rel1, W1_rel2])
    b1 = jnp.stack([b1_rel0, b1_rel1, b1_rel2])
    w2 = jnp.stack([W2_rel0, W2_rel1, W2_rel2])
    b2 = jnp.stack([b2_rel0, b2_rel1, b2_rel2])

    g1, cnt1 = _seg(feat, srcp, dstp, ones16, zeros128, zeros16)
    h1 = _tc_relu(g1, cnt1, w1, b1)
    g2, _ = _seg(h1, srcp, dstp, ones16, zeros128, zeros16)
    out = _tc_lin(g2, cnt1, w2, b2)
    return out


# speedup vs baseline: 1.6367x; 1.0027x over previous
import functools

import jax
import jax.numpy as jnp
from jax import lax
from jax.experimental import pallas as pl
from jax.experimental.pallas import tpu as pltpu
from jax.experimental.pallas import tpu_sc as plsc

NN = 10000
EE = 160000
DD = 128
NREL = 3
NC = 2
NS = 16
CHUNK = 128
CH = 80
EPAD = NS * CH * CHUNK
HALF = NN // NC
ACC_H = 5001
OUT_H = 5000
CNTW = 16
ZCH = 40
KPT = 3


def _seg_body(table, src_h, dst_h, ones_h, zeros_h, zeros16_h,
              g_out, cnt_out, src_v, dst_v, rows0_v, rows1_v, ones_v, zero_v,
              zero16_v, acc, cnt, sem0, sem1, sems0, sems1, semc):
    c = lax.axis_index("c")
    s = lax.axis_index("s")

    pltpu.sync_copy(zeros_h, zero_v)
    pltpu.sync_copy(ones_h, ones_v)
    pltpu.sync_copy(zeros16_h, zero16_v)

    def row_base(k):
        return jnp.minimum((s + NS * k) * CHUNK, OUT_H - CHUNK)

    def per_etype(e, carry):
        def zero_step(k, cc):
            base = row_base(k)
            pltpu.sync_copy(zero_v, acc.at[pl.ds(base, CHUNK)])
            pltpu.sync_copy(zero16_v, cnt.at[pl.ds(base, CHUNK)])
            return cc

        lax.fori_loop(0, KPT, zero_step, 0)
        plsc.subcore_barrier()

        pltpu.sync_copy(
            src_h.at[pl.ds((e * NS + s) * CH * CHUNK, CH * CHUNK)], src_v)
        pltpu.sync_copy(dst_h.at[pl.ds(((c * NREL + e) * NS + s) * CH, CH)],
                        dst_v)

        pltpu.async_copy(
            table.at[src_v.at[pl.ds(0, CHUNK)]], rows0_v, sem0)
        pltpu.async_copy(
            table.at[src_v.at[pl.ds(CHUNK, CHUNK)]], rows1_v, sem1)

        def pair_step(p, cc):
            j0 = 2 * p
            j1 = 2 * p + 1
            pltpu.make_async_copy(
                table.at[src_v.at[pl.ds(j0 * CHUNK, CHUNK)]], rows0_v,
                sem0).wait()
            pltpu.async_copy(rows0_v, acc.at[dst_v.at[j0]], sems0, add=True)
            pltpu.async_copy(ones_v, cnt.at[dst_v.at[j0]], semc, add=True)
            pltpu.make_async_copy(
                table.at[src_v.at[pl.ds(j1 * CHUNK, CHUNK)]], rows1_v,
                sem1).wait()
            pltpu.async_copy(rows1_v, acc.at[dst_v.at[j1]], sems1, add=True)
            pltpu.make_async_copy(ones_v, cnt.at[dst_v.at[j0]],
                                  semc).wait()
            pltpu.async_copy(ones_v, cnt.at[dst_v.at[j1]], semc, add=True)
            pltpu.make_async_copy(
                rows0_v, acc.at[dst_v.at[j0]], sems0).wait()

            @pl.when(p + 1 < CH // 2)
            def _():
                pltpu.async_copy(
                    table.at[src_v.at[pl.ds((j0 + 2) * CHUNK, CHUNK)]],
                    rows0_v, sem0)

            pltpu.make_async_copy(
                rows1_v, acc.at[dst_v.at[j1]], sems1).wait()
            pltpu.make_async_copy(ones_v, cnt.at[dst_v.at[j1]],
                                  semc).wait()

            @pl.when(p + 1 < CH // 2)
            def _():
                pltpu.async_copy(
                    table.at[src_v.at[pl.ds((j1 + 2) * CHUNK, CHUNK)]],
                    rows1_v, sem1)
            return cc

        lax.fori_loop(0, CH // 2, pair_step, 0)
        plsc.subcore_barrier()

        def dump_step(k, cc):
            base = row_base(k)
            pltpu.sync_copy(acc.at[pl.ds(base, CHUNK)],
                            g_out.at[e, c, pl.ds(base, CHUNK)])
            pltpu.sync_copy(cnt.at[pl.ds(base, CHUNK)],
                            cnt_out.at[e, c, pl.ds(base, CHUNK)])
            return cc

        lax.fori_loop(0, KPT, dump_step, 0)
        plsc.subcore_barrier()
        return carry

    lax.fori_loop(0, NREL, per_etype, 0)


def _make_seg_kernel():
    mesh = plsc.VectorSubcoreMesh(core_axis_name="c", subcore_axis_name="s")
    out_type = (
        jax.ShapeDtypeStruct((NREL, NC, OUT_H, DD), jnp.bfloat16),
        jax.ShapeDtypeStruct((NREL, NC, OUT_H, CNTW), jnp.float32),
    )
    scratch = [
        pltpu.VMEM((CH * CHUNK,), jnp.int32),
        pltpu.VMEM((CH, CHUNK), jnp.int32),
        pltpu.VMEM((CHUNK, DD), jnp.bfloat16),
        pltpu.VMEM((CHUNK, DD), jnp.bfloat16),
        pltpu.VMEM((CHUNK, CNTW), jnp.float32),
        pltpu.VMEM((CHUNK, DD), jnp.bfloat16),
        pltpu.VMEM((CHUNK, CNTW), jnp.float32),
        pltpu.VMEM_SHARED((ACC_H, DD), jnp.bfloat16),
        pltpu.VMEM_SHARED((ACC_H, CNTW), jnp.float32),
        pltpu.SemaphoreType.DMA,
        pltpu.SemaphoreType.DMA,
        pltpu.SemaphoreType.DMA,
        pltpu.SemaphoreType.DMA,
        pltpu.SemaphoreType.DMA,
    ]
    return pl.kernel(
        _seg_body,
        out_type=out_type,
        mesh=mesh,
        scratch_types=scratch,
        compiler_params=pltpu.CompilerParams(use_tc_tiling_on_sc=False),
    )


_seg = _make_seg_kernel()

BR = 1000
NBLK = HALF // BR


def _tc_body(relu, gp, cp, w, b, out):
    g = gp[:, 0].astype(jnp.float32)
    cnt = cp[:, 0, :, 0]
    denom = jnp.maximum(cnt, 1.0)
    ind = (cnt > 0.0).astype(jnp.float32)
    acc = jnp.zeros((BR, DD), jnp.float32)
    for e in range(NREL):
        gn = g[e] * (1.0 / denom[e])[:, None]
        acc = acc + jnp.dot(gn, w[e], preferred_element_type=jnp.float32)
        acc = acc + ind[e][:, None] * b[e][None, :]
    if relu:
        acc = jnp.maximum(acc, 0.0)
    out[...] = acc.astype(out.dtype)


def _make_tc_layer(relu, out_dtype):
    return pl.pallas_call(
        functools.partial(_tc_body, relu),
        grid=(NN // BR,),
        in_specs=[
            pl.BlockSpec((NREL, 1, BR, DD),
                         lambda i: (0, i // NBLK, i % NBLK, 0)),
            pl.BlockSpec((NREL, 1, BR, CNTW),
                         lambda i: (0, i // NBLK, i % NBLK, 0)),
            pl.BlockSpec((NREL, DD, DD), lambda i: (0, 0, 0)),
            pl.BlockSpec((NREL, DD), lambda i: (0, 0)),
        ],
        out_specs=pl.BlockSpec((BR, DD), lambda i: (i, 0)),
        out_shape=jax.ShapeDtypeStruct((NN, DD), out_dtype),
    )


_tc_relu = _make_tc_layer(True, jnp.bfloat16)
_tc_lin = _make_tc_layer(False, jnp.float32)


def kernel(feat, edge_index_rel0, edge_index_rel1, edge_index_rel2,
           W1_rel0, b1_rel0, W1_rel1, b1_rel1, W1_rel2, b1_rel2,
           W2_rel0, b2_rel0, W2_rel1, b2_rel1, W2_rel2, b2_rel2):
    src = jnp.stack([edge_index_rel0[0], edge_index_rel1[0], edge_index_rel2[0]])
    dst = jnp.stack([edge_index_rel0[1], edge_index_rel1[1], edge_index_rel2[1]])
    pad = EPAD - EE
    srcp = jnp.concatenate(
        [src, jnp.zeros((NREL, pad), jnp.int32)], axis=1
    ).reshape(NREL * NS * CH * CHUNK)
    dstp_cores = []
    dpad = jnp.concatenate([dst, jnp.full((NREL, pad), NN, jnp.int32)], axis=1)
    for c in range(NC):
        dl = dpad - c * HALF
        dl = jnp.where((dl >= 0) & (dl < HALF), dl, HALF)
        dstp_cores.append(dl.reshape(NREL * NS * CH, CHUNK))
    dstp = jnp.concatenate(dstp_cores)

    ones16 = jnp.ones((CHUNK, CNTW), jnp.float32)
    zeros128 = jnp.zeros((CHUNK, DD), jnp.bfloat16)
    zeros16 = jnp.zeros((CHUNK, CNTW), jnp.float32)
    feat = feat.astype(jnp.bfloat16)

    w1 = jnp.stack([W1_rel0, W1_rel1, W1_rel2])
    b1 = jnp.stack([b1_rel0, b1_rel1, b1_rel2])
    w2 = jnp.stack([W2_rel0, W2_rel1, W2_rel2])
    b2 = jnp.stack([b2_rel0, b2_rel1, b2_rel2])

    g1, cnt1 = _seg(feat, srcp, dstp, ones16, zeros128, zeros16)
    h1 = _tc_relu(g1, cnt1, w1, b1)
    g2, _ = _seg(h1, srcp, dstp, ones16, zeros128, zeros16)
    out = _tc_lin(g2, cnt1, w2, b2)
    return out
